# Initial kernel scaffold; baseline (speedup 1.0000x reference)
import jax
import jax.numpy as jnp
from jax.experimental import pallas as pl


def kernel(bytes, patch_ids, W, b):
    raise NotImplementedError("write your pallas kernel here")


# speedup vs baseline: 6.0829x; 6.0829x over previous
import functools

import jax
import jax.numpy as jnp
from jax.experimental import pallas as pl
from jax.experimental.pallas import tpu as pltpu

_NUM_PATCHES = 2048
_ST = 1024


def _fused_body(ids_ref, bytes_ref, w_ref, b_ref, be_out, pe_out, sums, counts,
                *, ns, num_patches):
    s = pl.program_id(1)
    h32 = bytes_ref[0]
    hbf = h32.astype(jnp.bfloat16)
    be_out[0] = hbf

    ids = ids_ref[0, 0]
    patches = jax.lax.broadcasted_iota(jnp.int32, (num_patches, hbf.shape[0]), 0)
    oh = (patches == ids[None, :]).astype(jnp.bfloat16)
    partial = jax.lax.dot_general(
        oh, hbf, (((1,), (0,)), ((), ())),
        preferred_element_type=jnp.float32)
    cpart = jnp.sum(oh.astype(jnp.float32), axis=1)[:, None]

    @pl.when(s == 0)
    def _():
        sums[...] = partial
        counts[...] = cpart

    @pl.when(s != 0)
    def _():
        sums[...] += partial
        counts[...] += cpart

    @pl.when(s == ns - 1)
    def _():
        cnt = jnp.maximum(counts[...], 1.0)
        mean = (sums[...] / cnt).astype(jnp.bfloat16).astype(jnp.float32)
        pe_out[0] = jax.lax.dot_general(
            mean, w_ref[...], (((1,), (0,)), ((), ())),
            preferred_element_type=jnp.float32) + b_ref[0][None, :]


def kernel(bytes, patch_ids, W, b):
    B, S, D = bytes.shape
    GD = W.shape[1]
    NP = _NUM_PATCHES
    ST = min(_ST, S)
    ns = S // ST
    ids3 = patch_ids.reshape(B * ns, 1, ST).astype(jnp.int32)
    b2 = b.reshape(1, GD)

    body = functools.partial(_fused_body, ns=ns, num_patches=NP)
    be, pe = pl.pallas_call(
        body,
        grid=(B, ns),
        in_specs=[
            pl.BlockSpec((1, 1, ST), lambda bb, ss: (bb * ns + ss, 0, 0)),
            pl.BlockSpec((1, ST, D), lambda bb, ss: (bb, ss, 0)),
            pl.BlockSpec((D, GD), lambda bb, ss: (0, 0)),
            pl.BlockSpec((1, GD), lambda bb, ss: (0, 0)),
        ],
        out_specs=[
            pl.BlockSpec((1, ST, D), lambda bb, ss: (bb, ss, 0)),
            pl.BlockSpec((1, NP, GD), lambda bb, ss: (bb, 0, 0)),
        ],
        out_shape=[
            jax.ShapeDtypeStruct((B, S, D), jnp.bfloat16),
            jax.ShapeDtypeStruct((B, NP, GD), jnp.float32),
        ],
        scratch_shapes=[
            pltpu.VMEM((NP, D), jnp.float32),
            pltpu.VMEM((NP, 1), jnp.float32),
        ],
    )(ids3, bytes, W, b2)
    return (be, pe)
